# Initial kernel scaffold; baseline (speedup 1.0000x reference)
import jax
import jax.numpy as jnp
from jax.experimental import pallas as pl


def kernel(logits):
    raise NotImplementedError("write your pallas kernel here")


# speedup vs baseline: 166.4381x; 166.4381x over previous
import functools

import jax
import jax.numpy as jnp
from jax import lax
from jax.experimental import pallas as pl
from jax.experimental.pallas import tpu as pltpu
from jax.experimental.pallas import tpu_sc as plsc

L = 16
NBUF = 2
UNROLL = 8


def _lane_top2_insert(m1, m2, v):
    mn = jnp.minimum(m1, v)
    return jnp.maximum(m1, v), jnp.maximum(m2, mn)


def _pair_merge(a, b):
    a1, a2 = a
    b1, b2 = b
    hi = jnp.maximum(a1, b1)
    mid = jnp.minimum(a1, b1)
    lo = jnp.maximum(jnp.maximum(a2, b2), mid)
    return hi, lo


def _make_gap_kernel(n_rows, V, n_workers, rows_per_worker):
    mesh = plsc.VectorSubcoreMesh(core_axis_name="c", subcore_axis_name="s")
    num_cores = mesh.num_cores

    @functools.partial(
        pl.kernel,
        out_type=jax.ShapeDtypeStruct((n_workers, L), jnp.float32),
        mesh=mesh,
        compiler_params=pltpu.CompilerParams(needs_layout_passes=False),
        scratch_types=[
            pltpu.VMEM((NBUF, V), jnp.float32),
            pltpu.VMEM((L,), jnp.float32),
            pltpu.SemaphoreType.DMA,
            pltpu.SemaphoreType.DMA,
        ],
    )
    def gap_kernel(x_hbm, out_hbm, buf, outbuf, sem0, sem1):
        sems = (sem0, sem1)
        wid = lax.axis_index("s") * num_cores + lax.axis_index("c")
        base = wid * rows_per_worker

        for b in range(NBUF):
            pltpu.make_async_copy(x_hbm.at[base + b], buf.at[b], sems[b]).start()

        neg = jnp.full((L,), -3.0e38, jnp.float32)

        def row_top2(b):
            def inner(i, carry):
                a1, a2, b1, b2, c1, c2, d1, d2 = carry
                o = i * (UNROLL * L)
                v0 = buf[b, pl.ds(o + 0 * L, L)]
                v1 = buf[b, pl.ds(o + 1 * L, L)]
                v2 = buf[b, pl.ds(o + 2 * L, L)]
                v3 = buf[b, pl.ds(o + 3 * L, L)]
                v4 = buf[b, pl.ds(o + 4 * L, L)]
                v5 = buf[b, pl.ds(o + 5 * L, L)]
                v6 = buf[b, pl.ds(o + 6 * L, L)]
                v7 = buf[b, pl.ds(o + 7 * L, L)]
                a1, a2 = _lane_top2_insert(a1, a2, v0)
                b1, b2 = _lane_top2_insert(b1, b2, v1)
                c1, c2 = _lane_top2_insert(c1, c2, v2)
                d1, d2 = _lane_top2_insert(d1, d2, v3)
                a1, a2 = _lane_top2_insert(a1, a2, v4)
                b1, b2 = _lane_top2_insert(b1, b2, v5)
                c1, c2 = _lane_top2_insert(c1, c2, v6)
                d1, d2 = _lane_top2_insert(d1, d2, v7)
                return a1, a2, b1, b2, c1, c2, d1, d2

            a1, a2, b1, b2, c1, c2, d1, d2 = lax.fori_loop(
                0, V // (UNROLL * L), inner, (neg,) * 8)
            m1, m2 = _pair_merge(
                _pair_merge((a1, a2), (b1, b2)),
                _pair_merge((c1, c2), (d1, d2)))

            top1 = jnp.max(m1)
            eq = m1 == top1
            firsts = jnp.cumsum(eq.astype(jnp.int32))
            first = jnp.logical_and(eq, firsts == 1)
            merged = jnp.where(first, m2, m1)
            top2 = jnp.max(merged)
            return top1 - top2

        def outer(g, acc):
            for b in range(NBUF):
                pltpu.make_async_copy(x_hbm.at[base], buf.at[b], sems[b]).wait()
                gap = row_top2(b)
                acc = acc + gap
                nxt = g * NBUF + b + NBUF

                @pl.when(nxt < rows_per_worker)
                def _():
                    pltpu.make_async_copy(
                        x_hbm.at[base + nxt], buf.at[b], sems[b]).start()
            return acc

        acc = lax.fori_loop(0, rows_per_worker // NBUF, outer,
                            jnp.zeros((L,), jnp.float32))
        outbuf[...] = acc * jnp.float32(1.0 / rows_per_worker)
        pltpu.sync_copy(outbuf, out_hbm.at[wid])

    return gap_kernel


def kernel(logits):
    B, S, V = logits.shape
    n_rows = B * S
    n_workers = 2 * 16
    rows_per_worker = n_rows // n_workers
    x = logits.reshape(n_rows, V)
    out = _make_gap_kernel(n_rows, V, n_workers, rows_per_worker)(x)
    gap = out[:, 0]
    avg_gap = jnp.mean(gap)
    should_exit = avg_gap >= jnp.float32(7.5)
    return gap, avg_gap, should_exit


# speedup vs baseline: 305.9291x; 1.8381x over previous
import functools

import jax
import jax.numpy as jnp
from jax import lax
from jax.experimental import pallas as pl
from jax.experimental.pallas import tpu as pltpu
from jax.experimental.pallas import tpu_sc as plsc

L = 16
NBUF = 3
NPAIR = 4
UNROLL = 16


def _lane_top2_insert(m1, m2, v):
    mn = jnp.minimum(m1, v)
    return jnp.maximum(m1, v), jnp.maximum(m2, mn)


def _pair_merge(a, b):
    a1, a2 = a
    b1, b2 = b
    hi = jnp.maximum(a1, b1)
    mid = jnp.minimum(a1, b1)
    lo = jnp.maximum(jnp.maximum(a2, b2), mid)
    return hi, lo


def _make_gap_kernel(n_rows, V, n_workers, rows_per_worker):
    mesh = plsc.VectorSubcoreMesh(core_axis_name="c", subcore_axis_name="s")
    num_cores = mesh.num_cores

    @functools.partial(
        pl.kernel,
        out_type=jax.ShapeDtypeStruct((n_workers, L), jnp.float32),
        mesh=mesh,
        compiler_params=pltpu.CompilerParams(needs_layout_passes=False),
        scratch_types=[
            pltpu.VMEM((V,), jnp.float32),
            pltpu.VMEM((V,), jnp.float32),
            pltpu.VMEM((V,), jnp.float32),
            pltpu.VMEM((L,), jnp.float32),
            pltpu.SemaphoreType.DMA,
            pltpu.SemaphoreType.DMA,
            pltpu.SemaphoreType.DMA,
        ],
    )
    def gap_kernel(x_hbm, out_hbm, buf0, buf1, buf2, outbuf, sem0, sem1, sem2):
        bufs = (buf0, buf1, buf2)
        sems = (sem0, sem1, sem2)
        wid = lax.axis_index("s") * num_cores + lax.axis_index("c")
        base = wid * rows_per_worker

        for b in range(NBUF):
            pltpu.make_async_copy(x_hbm.at[base + b], bufs[b], sems[b]).start()

        neg = jnp.full((L,), -3.0e38, jnp.float32)

        def row_top2(b):
            def inner(i, carry):
                pairs = [list(carry[2 * j:2 * j + 2]) for j in range(NPAIR)]
                o = i * (UNROLL * L)
                for k in range(UNROLL):
                    v = bufs[b][pl.ds(o + k * L, L)]
                    j = k % NPAIR
                    pairs[j][0], pairs[j][1] = _lane_top2_insert(
                        pairs[j][0], pairs[j][1], v)
                return tuple(x for p in pairs for x in p)

            res = lax.fori_loop(0, V // (UNROLL * L), inner, (neg,) * (2 * NPAIR))
            pairs = [(res[2 * j], res[2 * j + 1]) for j in range(NPAIR)]
            while len(pairs) > 1:
                pairs = [_pair_merge(pairs[i], pairs[i + 1])
                         for i in range(0, len(pairs), 2)]
            m1, m2 = pairs[0]

            top1 = jnp.max(m1)
            eq = m1 == top1
            firsts = jnp.cumsum(eq.astype(jnp.int32))
            first = jnp.logical_and(eq, firsts == 1)
            merged = jnp.where(first, m2, m1)
            top2 = jnp.max(merged)
            return top1 - top2

        def outer(g, acc):
            for b in range(NBUF):
                pltpu.make_async_copy(x_hbm.at[base], bufs[b], sems[b]).wait()
                gap = row_top2(b)
                acc = acc + gap
                nxt = g * NBUF + b + NBUF

                @pl.when(nxt < rows_per_worker)
                def _():
                    pltpu.make_async_copy(
                        x_hbm.at[base + nxt], bufs[b], sems[b]).start()
            return acc

        acc = lax.fori_loop(0, rows_per_worker // NBUF, outer,
                            jnp.zeros((L,), jnp.float32))
        for b in range(rows_per_worker % NBUF):
            pltpu.make_async_copy(x_hbm.at[base], bufs[b], sems[b]).wait()
            acc = acc + row_top2(b)
        outbuf[...] = acc * jnp.float32(1.0 / rows_per_worker)
        pltpu.sync_copy(outbuf, out_hbm.at[wid])

    return gap_kernel


def kernel(logits):
    B, S, V = logits.shape
    n_rows = B * S
    n_workers = 2 * 16
    rows_per_worker = n_rows // n_workers
    x = logits.reshape(n_rows, V)
    out = _make_gap_kernel(n_rows, V, n_workers, rows_per_worker)(x)
    gap = out[:, 0]
    avg_gap = jnp.mean(gap)
    should_exit = avg_gap >= jnp.float32(7.5)
    return gap, avg_gap, should_exit
